# Initial kernel scaffold; baseline (speedup 1.0000x reference)
import jax
import jax.numpy as jnp
from jax.experimental import pallas as pl


def kernel(data, phy_edge_index, net_edge_index, mul_edge_index, mul_emb, W1, b1, gamma1, beta1, W2, b2, gamma2, beta2, bn_g, bn_b, lin_W, lin_b, conv_W, conv_b):
    raise NotImplementedError("write your pallas kernel here")


# speedup vs baseline: 78.8847x; 78.8847x over previous
import jax
import jax.numpy as jnp
from jax.experimental import pallas as pl
from jax.experimental.pallas import tpu as pltpu

N = 500
B = 64
F = 60
C1 = 16
C2 = 64
TOPK = 20
INV21 = 1.0 / 21.0
INV_EPS = 1.0 / (1.0 + 1e-5) ** 0.5


def _graph_kernel(emb_ref, a_ref, cos_ref):
    w = emb_ref[:]
    inv_nrm = jax.lax.rsqrt(jnp.sum(w * w, axis=1, keepdims=True))
    wn = w * inv_nrm
    cos_ref[:] = jax.lax.dot_general(
        wn, wn, (((1,), (1,)), ((), ())),
        preferred_element_type=jnp.float32)
    col = jax.lax.broadcasted_iota(jnp.int32, (N, N), 1)
    row = jax.lax.broadcasted_iota(jnp.int32, (N, N), 0)
    a_ref[:] = jnp.where(row == col, INV21, 0.0).astype(jnp.float32)

    def body(_, carry):
        cos = cos_ref[:]
        m = jnp.max(cos, axis=1, keepdims=True)
        cand = jnp.where(cos == m, col, N)
        amin = jnp.min(cand, axis=1, keepdims=True)
        onehot = col == amin
        a_ref[:] = a_ref[:] + jnp.where(onehot, INV21, 0.0)
        cos_ref[:] = jnp.where(onehot, -jnp.inf, cos)
        return carry

    jax.lax.fori_loop(0, TOPK, body, 0)


def _feat_kernel(xnm_ref, xbm_ref, w1_ref, convw_ref, convb_ref,
                 h1_ref, mulx_ref):
    h1_ref[:] = jnp.dot(xnm_ref[:], w1_ref[:],
                        preferred_element_type=jnp.float32)
    mulx_ref[:] = jax.lax.dot_general(
        xbm_ref[:], convw_ref[:], (((1,), (1,)), ((), ())),
        preferred_element_type=jnp.float32) + convb_ref[:]


def _agg1_kernel(a_ref, h_ref, s_ref, t_ref, y_ref):
    ag = jnp.dot(a_ref[:], h_ref[:], preferred_element_type=jnp.float32)
    y_ref[:] = jnp.maximum(ag * s_ref[:] + t_ref[:], 0.0)


def _h2_kernel(y_ref, w2_ref, h2_ref):
    h2_ref[:] = jnp.dot(y_ref[:], w2_ref[:],
                        preferred_element_type=jnp.float32)


def _agg2_kernel(a_ref, h_ref, s_ref, t_ref, z_ref):
    ag = jnp.dot(a_ref[:], h_ref[:], preferred_element_type=jnp.float32)
    z_ref[:] = ag * s_ref[:] + t_ref[:]


def _out_kernel(z_ref, emb_ref, sg_ref, sb_ref, linw_ref, linb_ref, out_ref):
    z = z_ref[:]
    m = jnp.max(z, axis=1, keepdims=True)
    e = jnp.exp(z - m)
    lse = jnp.log(jnp.sum(e, axis=1, keepdims=True)) + m
    o = (z - lse) * emb_ref[:]
    o = jnp.maximum(o * sg_ref[:] + sb_ref[:], 0.0)
    out_ref[:] = jax.lax.dot_general(
        o, linw_ref[:], (((1,), (1,)), ((), ())),
        preferred_element_type=jnp.float32) + linb_ref[:]


def kernel(data, phy_edge_index, net_edge_index, mul_edge_index, mul_emb,
           W1, b1, gamma1, beta1, W2, b2, gamma2, beta2,
           bn_g, bn_b, lin_W, lin_b, conv_W, conv_b):
    f32 = jnp.float32
    xnm = data.transpose(1, 0, 2).reshape(N * B, F)
    xbm = data.reshape(N * B, F)

    s1 = gamma1 * INV_EPS
    t1 = b1 * s1 + beta1
    s1t = jnp.tile(s1, B).reshape(1, B * C1)
    t1t = jnp.tile(t1, B).reshape(1, B * C1)
    s2 = gamma2 * INV_EPS
    t2 = b2 * s2 + beta2
    s2t = jnp.tile(s2, B).reshape(1, B * C2)
    t2t = jnp.tile(t2, B).reshape(1, B * C2)
    sg = (bn_g * INV_EPS).reshape(1, C2)
    sb = bn_b.reshape(1, C2)
    convb = conv_b.reshape(1, C2)
    linb = lin_b.reshape(1, C2)
    emb_exp = jnp.broadcast_to(mul_emb[:, None, :], (N, B, C2)).reshape(N * B, C2)

    a = pl.pallas_call(
        _graph_kernel,
        out_shape=jax.ShapeDtypeStruct((N, N), f32),
        scratch_shapes=[pltpu.VMEM((N, N), f32)],
    )(mul_emb)

    grid_m = 8
    blk = (N * B) // grid_m
    full2 = lambda shape: pl.BlockSpec(shape, lambda i: (0, 0))
    rows = lambda w: pl.BlockSpec((blk, w), lambda i: (i, 0))

    h1, mulx = pl.pallas_call(
        _feat_kernel,
        grid=(grid_m,),
        in_specs=[rows(F), rows(F), full2((F, C1)), full2((C2, F)),
                  full2((1, C2))],
        out_specs=[rows(C1), rows(C2)],
        out_shape=[jax.ShapeDtypeStruct((N * B, C1), f32),
                   jax.ShapeDtypeStruct((N * B, C2), f32)],
    )(xnm, xbm, W1, conv_W, convb)

    y1 = pl.pallas_call(
        _agg1_kernel,
        out_shape=jax.ShapeDtypeStruct((N, B * C1), f32),
    )(a, h1.reshape(N, B * C1), s1t, t1t)

    h2 = pl.pallas_call(
        _h2_kernel,
        grid=(grid_m,),
        in_specs=[rows(C1), full2((C1, C2))],
        out_specs=rows(C2),
        out_shape=jax.ShapeDtypeStruct((N * B, C2), f32),
    )(y1.reshape(N * B, C1), W2)

    z = pl.pallas_call(
        _agg2_kernel,
        out_shape=jax.ShapeDtypeStruct((N, B * C2), f32),
    )(a, h2.reshape(N, B * C2), s2t, t2t)

    out_nm = pl.pallas_call(
        _out_kernel,
        grid=(grid_m,),
        in_specs=[rows(C2), rows(C2), full2((1, C2)), full2((1, C2)),
                  full2((C2, C2)), full2((1, C2))],
        out_specs=rows(C2),
        out_shape=jax.ShapeDtypeStruct((N * B, C2), f32),
    )(z.reshape(N * B, C2), emb_exp, sg, sb, lin_W, linb)

    out = out_nm.reshape(N, B, C2).transpose(1, 0, 2).reshape(N * B, C2)
    return out, mulx


# speedup vs baseline: 93.0178x; 1.1792x over previous
import jax
import jax.numpy as jnp
from jax.experimental import pallas as pl
from jax.experimental.pallas import tpu as pltpu

N = 500
B = 64
F = 60
C1 = 16
C2 = 64
TOPK = 20
INV21 = 1.0 / 21.0
INV_EPS = 1.0 / (1.0 + 1e-5) ** 0.5
FEAT_G = 8
OUT_G = 2
BF = jnp.bfloat16


def _feat_kernel(x_ref, wcat_ref, convb_ref, h1_ref, mulx_ref):
    wcat = wcat_ref[:]
    convb = convb_ref[:]
    for b in range(FEAT_G):
        xb = x_ref[b].astype(BF)
        hc = jnp.dot(xb, wcat, preferred_element_type=jnp.float32)
        h1_ref[b] = hc[:, :C1].astype(BF)
        mulx_ref[b * N:(b + 1) * N, :] = hc[:, C1:] + convb


def _mid_kernel(emb_ref, h1_ref, w2bd_ref, s1_ref, t1_ref, s2_ref, t2_ref,
                z_ref, cos_ref, a_ref):
    w = emb_ref[:]
    inv_nrm = jax.lax.rsqrt(jnp.sum(w * w, axis=1, keepdims=True))
    wn = w * inv_nrm
    cos_ref[:] = jax.lax.dot_general(
        wn, wn, (((1,), (1,)), ((), ())),
        preferred_element_type=jnp.float32)
    col = jax.lax.broadcasted_iota(jnp.int32, (N, N), 1)
    row = jax.lax.broadcasted_iota(jnp.int32, (N, N), 0)
    a_ref[:] = jnp.where(row == col, INV21, 0.0).astype(jnp.float32)

    def body(_, carry):
        cos = cos_ref[:]
        m = jnp.max(cos, axis=1, keepdims=True)
        cand = jnp.where(cos == m, col, N)
        amin = jnp.min(cand, axis=1, keepdims=True)
        onehot = col == amin
        a_ref[:] = a_ref[:] + jnp.where(onehot, INV21, 0.0)
        cos_ref[:] = jnp.where(onehot, -jnp.inf, cos)
        return carry

    jax.lax.fori_loop(0, TOPK, body, 0)
    a_bf = a_ref[:].astype(BF)

    h1 = jnp.concatenate([h1_ref[b] for b in range(B)], axis=1)
    ag1 = jnp.dot(a_bf, h1, preferred_element_type=jnp.float32)
    y1 = jnp.maximum(ag1 * s1_ref[:] + t1_ref[:], 0.0).astype(BF)

    h2 = jnp.dot(y1, w2bd_ref[:], preferred_element_type=jnp.float32)
    ag2 = jnp.dot(a_bf, h2.astype(BF), preferred_element_type=jnp.float32)
    z_ref[:] = ag2 * s2_ref[:] + t2_ref[:]


def _out_kernel(z_ref, emb_ref, sg_ref, sb_ref, linw_ref, linb_ref, out_ref):
    emb = emb_ref[:]
    sg = sg_ref[:]
    sb = sb_ref[:]
    linw = linw_ref[:]
    linb = linb_ref[:]
    z = z_ref[:]
    for j in range(OUT_G):
        zb = z[:, j * C2:(j + 1) * C2]
        m = jnp.max(zb, axis=1, keepdims=True)
        e = jnp.exp(zb - m)
        lse = jnp.log(jnp.sum(e, axis=1, keepdims=True)) + m
        o = (zb - lse) * emb
        o = jnp.maximum(o * sg + sb, 0.0).astype(BF)
        ob = jax.lax.dot_general(
            o, linw, (((1,), (1,)), ((), ())),
            preferred_element_type=jnp.float32) + linb
        out_ref[j * N:(j + 1) * N, :] = ob


def kernel(data, phy_edge_index, net_edge_index, mul_edge_index, mul_emb,
           W1, b1, gamma1, beta1, W2, b2, gamma2, beta2,
           bn_g, bn_b, lin_W, lin_b, conv_W, conv_b):
    f32 = jnp.float32

    wcat = jnp.concatenate([W1, conv_W.T], axis=1).astype(BF)
    convb = conv_b.reshape(1, C2)
    s1 = gamma1 * INV_EPS
    t1 = b1 * s1 + beta1
    s1t = jnp.tile(s1, B).reshape(1, B * C1)
    t1t = jnp.tile(t1, B).reshape(1, B * C1)
    s2 = gamma2 * INV_EPS
    t2 = b2 * s2 + beta2
    s2t = jnp.tile(s2, B).reshape(1, B * C2)
    t2t = jnp.tile(t2, B).reshape(1, B * C2)
    w2bd = jnp.kron(jnp.eye(B, dtype=f32), W2).astype(BF)
    sg = (bn_g * INV_EPS).reshape(1, C2)
    sb = bn_b.reshape(1, C2)
    linw = lin_W.astype(BF)
    linb = lin_b.reshape(1, C2)

    full = lambda shape: pl.BlockSpec(shape, lambda i: tuple(0 for _ in shape))

    h1, mulx = pl.pallas_call(
        _feat_kernel,
        grid=(B // FEAT_G,),
        in_specs=[pl.BlockSpec((FEAT_G, N, F), lambda i: (i, 0, 0)),
                  full((F, C1 + C2)), full((1, C2))],
        out_specs=[pl.BlockSpec((FEAT_G, N, C1), lambda i: (i, 0, 0)),
                   pl.BlockSpec((FEAT_G * N, C2), lambda i: (i, 0))],
        out_shape=[jax.ShapeDtypeStruct((B, N, C1), BF),
                   jax.ShapeDtypeStruct((N * B, C2), f32)],
    )(data, wcat, convb)

    z = pl.pallas_call(
        _mid_kernel,
        out_shape=jax.ShapeDtypeStruct((N, B * C2), f32),
        scratch_shapes=[pltpu.VMEM((N, N), f32), pltpu.VMEM((N, N), f32)],
    )(mul_emb, h1, w2bd, s1t, t1t, s2t, t2t)

    out = pl.pallas_call(
        _out_kernel,
        grid=(B // OUT_G,),
        in_specs=[pl.BlockSpec((N, OUT_G * C2), lambda i: (0, i)),
                  full((N, C2)), full((1, C2)), full((1, C2)),
                  full((C2, C2)), full((1, C2))],
        out_specs=pl.BlockSpec((OUT_G * N, C2), lambda i: (i, 0)),
        out_shape=jax.ShapeDtypeStruct((N * B, C2), f32),
    )(z, mul_emb, sg, sb, linw, linb)

    return out, mulx


# speedup vs baseline: 168.4074x; 1.8105x over previous
import jax
import jax.numpy as jnp
from jax.experimental import pallas as pl
from jax.experimental.pallas import tpu as pltpu

N = 500
B = 64
F = 60
C1 = 16
C2 = 64
TOPK = 20
INV21 = 1.0 / 21.0
INV_EPS = 1.0 / (1.0 + 1e-5) ** 0.5
BF = jnp.bfloat16


def _mgdn_kernel(x_ref, emb_ref, wcat_ref, convb_ref, s1_ref, t1_ref,
                 w2_ref, s2_ref, t2_ref, sg_ref, sb_ref, linw_ref,
                 linb_ref, out_ref, mulx_ref, cos_ref, a_ref):
    w = emb_ref[:]
    inv_nrm = jax.lax.rsqrt(jnp.sum(w * w, axis=1, keepdims=True))
    wn = w * inv_nrm
    cos_ref[:] = jax.lax.dot_general(
        wn, wn, (((1,), (1,)), ((), ())),
        preferred_element_type=jnp.float32)
    col = jax.lax.broadcasted_iota(jnp.int32, (N, N), 1)
    row = jax.lax.broadcasted_iota(jnp.int32, (N, N), 0)
    a_ref[:] = jnp.where(row == col, INV21, 0.0).astype(jnp.float32)

    def body(_, carry):
        cos = cos_ref[:]
        m = jnp.max(cos, axis=1, keepdims=True)
        cand = jnp.where(cos == m, col, N)
        amin = jnp.min(cand, axis=1, keepdims=True)
        onehot = col == amin
        a_ref[:] = a_ref[:] + jnp.where(onehot, INV21, 0.0)
        cos_ref[:] = jnp.where(onehot, -jnp.inf, cos)
        return carry

    jax.lax.fori_loop(0, TOPK, body, 0)
    a_bf = a_ref[:].astype(BF)

    wcat = wcat_ref[:]
    convb = convb_ref[:]
    h1_parts = []
    for b in range(B):
        xb = x_ref[b]
        hc = jnp.dot(xb, wcat, preferred_element_type=jnp.float32)
        h1_parts.append(hc[:, :C1].astype(BF))
        mulx_ref[b * N:(b + 1) * N, :] = hc[:, C1:] + convb

    h1 = jnp.concatenate(h1_parts, axis=1)
    ag1 = jnp.dot(a_bf, h1, preferred_element_type=jnp.float32)
    y1 = jnp.maximum(ag1 * s1_ref[:] + t1_ref[:], 0.0).astype(BF)

    w2 = w2_ref[:]
    h2 = jnp.concatenate(
        [jnp.dot(y1[:, b * C1:(b + 1) * C1], w2,
                 preferred_element_type=jnp.float32).astype(BF)
         for b in range(B)], axis=1)

    emb = emb_ref[:]
    sg = sg_ref[:]
    sb = sb_ref[:]
    linw = linw_ref[:]
    linb = linb_ref[:]
    HB = B // 2
    for half in range(2):
        lo = half * HB * C2
        ag2 = jnp.dot(a_bf, h2[:, lo:lo + HB * C2],
                      preferred_element_type=jnp.float32)
        for j in range(HB):
            b = half * HB + j
            zb = (ag2[:, j * C2:(j + 1) * C2]
                  * s2_ref[:, b * C2:(b + 1) * C2]
                  + t2_ref[:, b * C2:(b + 1) * C2])
            m = jnp.max(zb, axis=1, keepdims=True)
            e = jnp.exp(zb - m)
            lse = jnp.log(jnp.sum(e, axis=1, keepdims=True)) + m
            o = (zb - lse) * emb
            o = jnp.maximum(o * sg + sb, 0.0).astype(BF)
            ob = jax.lax.dot_general(
                o, linw, (((1,), (1,)), ((), ())),
                preferred_element_type=jnp.float32) + linb
            out_ref[b * N:(b + 1) * N, :] = ob


def kernel(data, phy_edge_index, net_edge_index, mul_edge_index, mul_emb,
           W1, b1, gamma1, beta1, W2, b2, gamma2, beta2,
           bn_g, bn_b, lin_W, lin_b, conv_W, conv_b):
    f32 = jnp.float32

    wcat = jnp.concatenate([W1, conv_W.T], axis=1).astype(BF)
    convb = conv_b.reshape(1, C2)
    s1 = gamma1 * INV_EPS
    t1 = b1 * s1 + beta1
    s1t = jnp.tile(s1, B).reshape(1, B * C1)
    t1t = jnp.tile(t1, B).reshape(1, B * C1)
    s2 = gamma2 * INV_EPS
    t2 = b2 * s2 + beta2
    s2t = jnp.tile(s2, B).reshape(1, B * C2)
    t2t = jnp.tile(t2, B).reshape(1, B * C2)
    w2bf = W2.astype(BF)
    sg = (bn_g * INV_EPS).reshape(1, C2)
    sb = bn_b.reshape(1, C2)
    linw = lin_W.astype(BF)
    linb = lin_b.reshape(1, C2)

    out, mulx = pl.pallas_call(
        _mgdn_kernel,
        out_shape=[jax.ShapeDtypeStruct((N * B, C2), f32),
                   jax.ShapeDtypeStruct((N * B, C2), f32)],
        scratch_shapes=[pltpu.VMEM((N, N), f32), pltpu.VMEM((N, N), f32)],
    )(data.astype(BF), mul_emb, wcat, convb, s1t, t1t, w2bf, s2t, t2t,
      sg, sb, linw, linb)

    return out, mulx


# speedup vs baseline: 187.5462x; 1.1136x over previous
import jax
import jax.numpy as jnp
from jax.experimental import pallas as pl
from jax.experimental.pallas import tpu as pltpu

N = 500
B = 64
F = 60
C1 = 16
C2 = 64
TOPK = 20
INV21 = 1.0 / 21.0
INV_EPS = 1.0 / (1.0 + 1e-5) ** 0.5
BF = jnp.bfloat16


CH = 8
NCH = B // CH


def _mgdn_kernel(x_hbm, emb_ref, w1_ref, b1_ref, gamma1_ref, beta1_ref,
                 w2_ref, b2_ref, gamma2_ref, beta2_ref, bng_ref, bnb_ref,
                 linw_ref, linb_ref, convw_ref, convb_ref,
                 out_ref, mulx_ref, cos_ref, a_ref, xbuf, xsem):
    def x_copy(c):
        return pltpu.make_async_copy(
            x_hbm.at[pl.ds(c * CH, CH)], xbuf.at[c % 2], xsem.at[c % 2])

    x_copy(0).start()

    w = emb_ref[:]
    inv_nrm = jax.lax.rsqrt(jnp.sum(w * w, axis=1, keepdims=True))
    wn = w * inv_nrm
    cos_ref[:] = jax.lax.dot_general(
        wn, wn, (((1,), (1,)), ((), ())),
        preferred_element_type=jnp.float32)
    col = jax.lax.broadcasted_iota(jnp.int32, (N, N), 1)
    row = jax.lax.broadcasted_iota(jnp.int32, (N, N), 0)
    a_ref[:] = jnp.where(row == col, INV21, 0.0).astype(jnp.float32)

    def body(_, carry):
        cos = cos_ref[:]
        m = jnp.max(cos, axis=1, keepdims=True)
        cand = jnp.where(cos == m, col, N)
        amin = jnp.min(cand, axis=1, keepdims=True)
        onehot = col == amin
        a_ref[:] = a_ref[:] + jnp.where(onehot, INV21, 0.0)
        cos_ref[:] = jnp.where(onehot, -jnp.inf, cos)
        return carry

    jax.lax.fori_loop(0, TOPK, body, 0)
    a_bf = a_ref[:].astype(BF)

    wcat = jnp.concatenate([w1_ref[:], convw_ref[:].T], axis=1).astype(BF)
    convb = convb_ref[:]
    s1 = gamma1_ref[:] * INV_EPS
    t1 = b1_ref[:] * s1 + beta1_ref[:]
    s1t = jnp.concatenate([s1] * B, axis=1)
    t1t = jnp.concatenate([t1] * B, axis=1)
    s2 = gamma2_ref[:] * INV_EPS
    t2 = b2_ref[:] * s2 + beta2_ref[:]
    sg = bng_ref[:] * INV_EPS
    sb = bnb_ref[:]
    w2 = w2_ref[:].astype(BF)
    linw = linw_ref[:].astype(BF)
    linb = linb_ref[:]

    h1_parts = []
    for c in range(NCH):
        if c + 1 < NCH:
            x_copy(c + 1).start()
        x_copy(c).wait()
        for i in range(CH):
            b = c * CH + i
            xb = xbuf[c % 2, i].astype(BF)
            hc = jnp.dot(xb, wcat, preferred_element_type=jnp.float32)
            h1_parts.append(hc[:, :C1].astype(BF))
            mulx_ref[b * N:(b + 1) * N, :] = hc[:, C1:] + convb

    h1 = jnp.concatenate(h1_parts, axis=1)
    HB = B // 2
    h2_parts = []
    for half in range(2):
        lo1 = half * HB * C1
        ag1 = jnp.dot(a_bf, h1[:, lo1:lo1 + HB * C1],
                      preferred_element_type=jnp.float32)
        y1 = jnp.maximum(ag1 * s1t[:, lo1:lo1 + HB * C1]
                         + t1t[:, lo1:lo1 + HB * C1], 0.0).astype(BF)
        h2_parts.extend(
            jnp.dot(y1[:, j * C1:(j + 1) * C1], w2,
                    preferred_element_type=jnp.float32).astype(BF)
            for j in range(HB))
    h2 = jnp.concatenate(h2_parts, axis=1)

    emb = emb_ref[:]
    for half in range(2):
        lo = half * HB * C2
        ag2 = jnp.dot(a_bf, h2[:, lo:lo + HB * C2],
                      preferred_element_type=jnp.float32)
        for j in range(HB):
            b = half * HB + j
            zb = ag2[:, j * C2:(j + 1) * C2] * s2 + t2
            m = jnp.max(zb, axis=1, keepdims=True)
            e = jnp.exp(zb - m)
            lse = jnp.log(jnp.sum(e, axis=1, keepdims=True)) + m
            o = (zb - lse) * emb
            o = jnp.maximum(o * sg + sb, 0.0).astype(BF)
            ob = jax.lax.dot_general(
                o, linw, (((1,), (1,)), ((), ())),
                preferred_element_type=jnp.float32) + linb
            out_ref[b * N:(b + 1) * N, :] = ob


def kernel(data, phy_edge_index, net_edge_index, mul_edge_index, mul_emb,
           W1, b1, gamma1, beta1, W2, b2, gamma2, beta2,
           bn_g, bn_b, lin_W, lin_b, conv_W, conv_b):
    f32 = jnp.float32
    row = lambda v: v.reshape(1, -1)

    out, mulx = pl.pallas_call(
        _mgdn_kernel,
        in_specs=[pl.BlockSpec(memory_space=pl.ANY)]
        + [pl.BlockSpec(memory_space=pltpu.MemorySpace.VMEM)] * 15,
        out_shape=[jax.ShapeDtypeStruct((N * B, C2), f32),
                   jax.ShapeDtypeStruct((N * B, C2), f32)],
        scratch_shapes=[pltpu.VMEM((N, N), f32), pltpu.VMEM((N, N), f32),
                        pltpu.VMEM((2, CH, N, F), f32),
                        pltpu.SemaphoreType.DMA((2,))],
    )(data, mul_emb, W1, row(b1), row(gamma1), row(beta1),
      W2, row(b2), row(gamma2), row(beta2), row(bn_g), row(bn_b),
      lin_W, row(lin_b), conv_W, row(conv_b))

    return out, mulx
